# baseline (speedup 1.0000x reference)
import functools

import jax
import jax.numpy as jnp
from jax import lax
from jax.experimental import pallas as pl
from jax.experimental.pallas import tpu as pltpu
from jax.experimental.pallas import tpu_sc as plsc

B, T, D = 16, 4096, 2048

_LN2 = 0.6931471805599453
_SQRT2 = 1.4142135623730951


def _neg_log(x):
    bits = lax.bitcast_convert_type(x, jnp.int32)
    e = lax.shift_right_arithmetic(bits, 23) - 127
    m = lax.bitcast_convert_type(
        (bits & jnp.int32(0x007FFFFF)) | jnp.int32(0x3F800000), jnp.float32
    )
    big = m > _SQRT2
    e = jnp.where(big, e + 1, e)
    m = jnp.where(big, m * 0.5, m)
    z = (m - 1.0) / (m + 1.0)
    z2 = z * z
    atanh = z * (1.0 + z2 * (1.0 / 3.0 + z2 * (1.0 / 5.0 + z2 * (1.0 / 7.0))))
    log_x = e.astype(jnp.float32) * _LN2 + 2.0 * atanh
    return -log_x


@functools.cache
def _make_sc_gather():
    mesh = plsc.VectorSubcoreMesh(core_axis_name="c", subcore_axis_name="s")

    @functools.partial(
        pl.kernel,
        mesh=mesh,
        out_type=[
            jax.ShapeDtypeStruct((B, D), jnp.float32),
            jax.ShapeDtypeStruct((B,), jnp.float32),
            jax.ShapeDtypeStruct((B,), jnp.float32),
        ],
        scratch_types=[
            pltpu.VMEM((B,), jnp.int32),
            pltpu.VMEM((B, D), jnp.float32),
            pltpu.VMEM((B,), jnp.float32),
            pltpu.VMEM((B,), jnp.float32),
            pltpu.SemaphoreType.DMA,
        ],
    )
    def sc_gather(table_hbm, lens_hbm, h_out, ln_out, nl_out,
                  idx_v, rows_v, ln_v, nl_v, sem):
        wid = lax.axis_index("s") * 2 + lax.axis_index("c")

        @pl.when(wid == 0)
        def _():
            pltpu.sync_copy(lens_hbm, idx_v)
            lens = idx_v[...]
            idx = lens - 1 + lax.iota(jnp.int32, B) * T
            idx_v[...] = idx
            pltpu.async_copy(table_hbm.at[idx_v], rows_v, sem).wait()
            pltpu.sync_copy(rows_v, h_out)
            lens_f = lens.astype(jnp.float32)
            ln = lens_f * (1.0 / 200.0)
            ln_v[...] = ln
            nl_v[...] = _neg_log(ln)
            pltpu.sync_copy(ln_v, ln_out)
            pltpu.sync_copy(nl_v, nl_out)

    return sc_gather


def kernel(payload, seq_lens):
    table = payload.reshape(B * T, D)
    lens32 = seq_lens.astype(jnp.int32)
    h, ln, nl = _make_sc_gather()(table, lens32)
    return jnp.concatenate([h, ln[:, None], nl[:, None]], axis=-1)
